# Initial kernel scaffold; baseline (speedup 1.0000x reference)
import jax
import jax.numpy as jnp
from jax.experimental import pallas as pl


def kernel(outputs, labels, domain_ids):
    raise NotImplementedError("write your pallas kernel here")


# speedup vs baseline: 2.6284x; 2.6284x over previous
import functools

import jax
import jax.numpy as jnp
from jax import lax
from jax.experimental import pallas as pl
from jax.experimental.pallas import tpu as pltpu
from jax.experimental.pallas import tpu_sc as plsc

N = 16384
NB = 128
L = 16
NS = 16
NPW = N // NS


def _build():
    mesh = plsc.VectorSubcoreMesh(
        core_axis_name="c", subcore_axis_name="s", num_cores=1
    )

    @functools.partial(
        pl.kernel,
        out_type=jax.ShapeDtypeStruct((L,), jnp.float32),
        mesh=mesh,
        compiler_params=pltpu.CompilerParams(needs_layout_passes=False),
        scratch_types=[
            pltpu.VMEM((NPW,), jnp.float32),
            pltpu.VMEM((NPW,), jnp.float32),
            pltpu.VMEM((NPW,), jnp.int32),
            pltpu.VMEM((NB,), jnp.float32),
            pltpu.VMEM((NB,), jnp.float32),
            pltpu.VMEM((2 * NB,), jnp.float32),
            pltpu.VMEM_SHARED((NS, 2 * NB), jnp.float32),
            pltpu.VMEM((NS, 2 * NB), jnp.float32),
            pltpu.VMEM((L,), jnp.float32),
        ],
    )
    def k(o_hbm, l_hbm, id_hbm, out_hbm, o_v, l_v, id_v, acc, cnt, pak, sh, buf, ov):
        s = lax.axis_index("s")
        base = s * NPW
        pltpu.sync_copy(o_hbm.at[pl.ds(base, NPW)], o_v)
        pltpu.sync_copy(l_hbm.at[pl.ds(base, NPW)], l_v)
        pltpu.sync_copy(id_hbm.at[pl.ds(base, NPW)], id_v)

        zeros = jnp.zeros((L,), jnp.float32)
        ones = jnp.ones((L,), jnp.float32)
        for j in range(NB // L):
            acc[pl.ds(j * L, L)] = zeros
            cnt[pl.ds(j * L, L)] = zeros

        for i in range(NPW // L):
            o = o_v[pl.ds(i * L, L)]
            t = l_v[pl.ds(i * L, L)]
            idx = id_v[pl.ds(i * L, L)]
            d = o - t
            plsc.addupdate_scatter(acc, [idx], d * d)
            plsc.addupdate_scatter(cnt, [idx], ones)

        for j in range(NB // L):
            pak[pl.ds(j * L, L)] = acc[pl.ds(j * L, L)]
            pak[pl.ds(NB + j * L, L)] = cnt[pl.ds(j * L, L)]
        pltpu.sync_copy(pak, sh.at[s])
        plsc.subcore_barrier()

        @pl.when(s == 0)
        def _():
            pltpu.sync_copy(sh, buf)
            sum_mse = zeros
            ndom = zeros
            for j in range(NB // L):
                ta = zeros
                tc = zeros
                for r in range(NS):
                    ta = ta + buf[r, pl.ds(j * L, L)]
                    tc = tc + buf[r, pl.ds(NB + j * L, L)]
                present = tc > 0.0
                safe = jnp.where(present, tc, ones)
                sum_mse = sum_mse + jnp.where(present, ta / safe, zeros)
                ndom = ndom + jnp.where(present, ones, zeros)
            total = jnp.full((L,), jnp.sum(sum_mse), jnp.float32)
            nd = jnp.full((L,), jnp.sum(ndom), jnp.float32)
            ov[...] = total / nd
            pltpu.sync_copy(ov, out_hbm)

    return k


_KERNEL = _build()


@jax.jit
def kernel(outputs, labels, domain_ids):
    res = _KERNEL(outputs, labels, domain_ids.astype(jnp.int32))
    return res[0]


# speedup vs baseline: 2.8244x; 1.0746x over previous
import functools

import jax
import jax.numpy as jnp
from jax import lax
from jax.experimental import pallas as pl
from jax.experimental.pallas import tpu as pltpu
from jax.experimental.pallas import tpu_sc as plsc

N = 16384
NB = 128
L = 16
NS = 16
NPW = N // NS


def _build():
    mesh = plsc.VectorSubcoreMesh(
        core_axis_name="c", subcore_axis_name="s", num_cores=1
    )

    @functools.partial(
        pl.kernel,
        out_type=jax.ShapeDtypeStruct((L,), jnp.float32),
        mesh=mesh,
        compiler_params=pltpu.CompilerParams(needs_layout_passes=False),
        scratch_types=[
            pltpu.VMEM((NPW,), jnp.float32),
            pltpu.VMEM((NPW,), jnp.float32),
            pltpu.VMEM((NPW,), jnp.int32),
            pltpu.VMEM((NB,), jnp.float32),
            pltpu.VMEM((NB,), jnp.float32),
            pltpu.VMEM_SHARED((NS, 2, NB), jnp.float32),
            pltpu.VMEM((NS, 2, NB), jnp.float32),
            pltpu.VMEM((L,), jnp.float32),
        ],
    )
    def k(o_hbm, l_hbm, id_hbm, out_hbm, o_v, l_v, id_v, acc, cnt, sh, buf, ov):
        s = lax.axis_index("s")
        base = s * NPW
        pltpu.sync_copy(o_hbm.at[pl.ds(base, NPW)], o_v)
        pltpu.sync_copy(l_hbm.at[pl.ds(base, NPW)], l_v)
        pltpu.sync_copy(id_hbm.at[pl.ds(base, NPW)], id_v)

        zeros = jnp.zeros((L,), jnp.float32)
        ones = jnp.ones((L,), jnp.float32)

        def zero_body(j, carry):
            acc[pl.ds(j * L, L)] = zeros
            cnt[pl.ds(j * L, L)] = zeros
            return carry

        lax.fori_loop(0, NB // L, zero_body, 0)

        def accum_body(i, carry):
            o = o_v[pl.ds(i * L, L)]
            t = l_v[pl.ds(i * L, L)]
            idx = id_v[pl.ds(i * L, L)]
            d = o - t
            plsc.addupdate_scatter(acc, [idx], d * d)
            plsc.addupdate_scatter(cnt, [idx], ones)
            return carry

        lax.fori_loop(0, NPW // L, accum_body, 0)

        pltpu.sync_copy(acc, sh.at[s, 0])
        pltpu.sync_copy(cnt, sh.at[s, 1])
        plsc.subcore_barrier()

        @pl.when(s == 0)
        def _():
            pltpu.sync_copy(sh, buf)

            def chunk_body(j, carry):
                sum_mse, ndom = carry

                def row_body(r, c2):
                    ta, tc = c2
                    ta = ta + buf[r, 0, pl.ds(j * L, L)]
                    tc = tc + buf[r, 1, pl.ds(j * L, L)]
                    return ta, tc

                ta, tc = lax.fori_loop(0, NS, row_body, (zeros, zeros))
                present = tc > 0.0
                safe = jnp.where(present, tc, ones)
                sum_mse = sum_mse + jnp.where(present, ta / safe, zeros)
                ndom = ndom + jnp.where(present, ones, zeros)
                return sum_mse, ndom

            sum_mse, ndom = lax.fori_loop(0, NB // L, chunk_body, (zeros, zeros))
            total = jnp.full((L,), jnp.sum(sum_mse), jnp.float32)
            nd = jnp.full((L,), jnp.sum(ndom), jnp.float32)
            ov[...] = total / nd
            pltpu.sync_copy(ov, out_hbm)

    return k


_KERNEL = _build()


@jax.jit
def kernel(outputs, labels, domain_ids):
    res = _KERNEL(outputs, labels, domain_ids.astype(jnp.int32))
    return res[0]


# speedup vs baseline: 3.0563x; 1.0821x over previous
import functools

import jax
import jax.numpy as jnp
from jax import lax
from jax.experimental import pallas as pl
from jax.experimental.pallas import tpu as pltpu
from jax.experimental.pallas import tpu_sc as plsc

N = 16384
NB = 128
L = 16
NS = 16
NPW = N // NS
NR = NB // L


def _build():
    mesh = plsc.VectorSubcoreMesh(
        core_axis_name="c", subcore_axis_name="s", num_cores=1
    )

    @functools.partial(
        pl.kernel,
        out_type=jax.ShapeDtypeStruct((L,), jnp.float32),
        mesh=mesh,
        compiler_params=pltpu.CompilerParams(needs_layout_passes=False),
        scratch_types=[
            pltpu.VMEM((NPW,), jnp.float32),
            pltpu.VMEM((NPW,), jnp.float32),
            pltpu.VMEM((NPW,), jnp.int32),
            pltpu.VMEM((NB,), jnp.float32),
            pltpu.VMEM((NB,), jnp.float32),
            pltpu.VMEM((L,), jnp.int32),
            pltpu.VMEM((2 * NR, L), jnp.float32),
            pltpu.VMEM_SHARED((2 * NR, L), jnp.float32),
            pltpu.VMEM((2 * NR, L), jnp.float32),
            pltpu.VMEM((L,), jnp.float32),
            pltpu.SemaphoreType.DMA,
        ],
    )
    def k(o_hbm, l_hbm, id_hbm, out_hbm,
          o_v, l_v, id_v, acc, cnt, idx16, pak, shz, buf, ov, sem):
        s = lax.axis_index("s")
        base = s * NPW
        co = pltpu.async_copy(o_hbm.at[pl.ds(base, NPW)], o_v, sem)
        cl = pltpu.async_copy(l_hbm.at[pl.ds(base, NPW)], l_v, sem)
        ci = pltpu.async_copy(id_hbm.at[pl.ds(base, NPW)], id_v, sem)

        zeros = jnp.zeros((L,), jnp.float32)
        ones = jnp.ones((L,), jnp.float32)
        idx16[...] = lax.iota(jnp.int32, L)

        def zero_body(j, carry):
            acc[pl.ds(j * L, L)] = zeros
            cnt[pl.ds(j * L, L)] = zeros
            pak[j, ...] = zeros
            pak[NR + j, ...] = zeros
            return carry

        lax.fori_loop(0, NR, zero_body, 0)

        @pl.when(s == 0)
        def _():
            pltpu.sync_copy(pak, shz)

        plsc.subcore_barrier()
        co.wait()
        cl.wait()
        ci.wait()

        def accum_body(i, carry):
            o = o_v[pl.ds(i * L, L)]
            t = l_v[pl.ds(i * L, L)]
            idx = id_v[pl.ds(i * L, L)]
            d = o - t
            plsc.addupdate_scatter(acc, [idx], d * d)
            plsc.addupdate_scatter(cnt, [idx], ones)
            return carry

        lax.fori_loop(0, NPW // L, accum_body, 0)

        def pack_body(j, carry):
            pak[j, ...] = acc[pl.ds(j * L, L)]
            pak[NR + j, ...] = cnt[pl.ds(j * L, L)]
            return carry

        lax.fori_loop(0, NR, pack_body, 0)

        pltpu.sync_copy(pak, shz.at[idx16], add=True)
        plsc.subcore_barrier()

        @pl.when(s == 0)
        def _():
            pltpu.sync_copy(shz, buf)

            def chunk_body(j, carry):
                sum_mse, ndom = carry
                ta = buf[j, ...]
                tc = buf[NR + j, ...]
                present = tc > 0.0
                safe = jnp.where(present, tc, ones)
                sum_mse = sum_mse + jnp.where(present, ta / safe, zeros)
                ndom = ndom + jnp.where(present, ones, zeros)
                return sum_mse, ndom

            sum_mse, ndom = lax.fori_loop(0, NR, chunk_body, (zeros, zeros))
            total = jnp.full((L,), jnp.sum(sum_mse), jnp.float32)
            nd = jnp.full((L,), jnp.sum(ndom), jnp.float32)
            ov[...] = total / nd
            pltpu.sync_copy(ov, out_hbm)

    return k


_KERNEL = _build()


@jax.jit
def kernel(outputs, labels, domain_ids):
    res = _KERNEL(outputs, labels, domain_ids.astype(jnp.int32))
    return res[0]
